# baseline (speedup 1.0000x reference)
import jax
import jax.numpy as jnp
from jax import lax
from jax.experimental import pallas as pl
from jax.experimental.pallas import tpu as pltpu
from jax.experimental.pallas import tpu_sc as plsc

_B = 4096
_L = 200
_D = 192
_DP = 256
_N = _B * _L
_NT = _L * _D
_BPS = 128
_IPS = _BPS * _L


def _gather_body(tbl_hbm, idx_hbm, out_hbm, idx_ref, xidx, s, sT, si, gs, ws):
    core = jax.lax.axis_index("core")
    sub = jax.lax.axis_index("subcore")
    sid = core * 16 + sub
    lane0 = sid * _BPS

    pltpu.async_copy(idx_hbm.at[pl.ds(sid * _IPS, _IPS)], idx_ref, si).wait()

    ii = lax.iota(jnp.int32, 16)
    rv = [ii + 16 * g for g in range(8)]
    bv = [ii * _L + 16 * g * _L for g in range(8)]

    def build_xidx(l, p):
        for g in range(8):
            v = plsc.load_gather(idx_ref, [bv[g] + l])
            xidx.at[p].at[pl.ds(16 * g, 16)][...] = v

    def start_gather(p):
        pltpu.async_copy(tbl_hbm.at[xidx.at[p]], s.at[p], gs.at[p])

    def wait_gather(p):
        pltpu.make_async_copy(tbl_hbm.at[xidx.at[p]], s.at[p], gs.at[p]).wait()

    def out_slc(l, h):
        return out_hbm.at[pl.ds(l * _D + 96 * h, 96), pl.ds(lane0, _BPS)]

    def wait_write(l, h):
        pltpu.make_async_copy(sT.at[h], out_slc(l, h), ws.at[h]).wait()

    def transpose_half(p, h):
        @pl.loop(0, 96)
        def _(dd):
            col = jnp.full((16,), 96 * h + dd, jnp.int32)
            row = jnp.full((16,), dd, jnp.int32)
            for g in range(8):
                v = plsc.load_gather(s.at[p], [rv[g], col])
                plsc.store_scatter(sT.at[h], [row, rv[g]], v)

    build_xidx(0, 0)
    start_gather(0)
    build_xidx(1, 1)
    start_gather(1)

    @pl.loop(0, _L // 2)
    def _(lp):
        for p in range(2):
            l = lp * 2 + p
            wait_gather(p)
            build_xidx(jnp.minimum(l + 2, _L - 1), p)
            for h in range(2):
                first_use = (lp == 0) & (p == 0)

                @pl.when(jnp.logical_not(first_use))
                def _():
                    wait_write(0, h)

                transpose_half(p, h)
                pltpu.async_copy(sT.at[h], out_slc(l, h), ws.at[h])
            start_gather(p)

    for p in range(2):
        wait_gather(p)
    for h in range(2):
        wait_write(0, h)


def kernel(x, emb1_weight, emb2_weight):
    table = jnp.concatenate(
        (
            emb1_weight,
            emb2_weight,
            jnp.zeros((emb1_weight.shape[0], _DP - _D), emb1_weight.dtype),
        ),
        axis=1,
    )
    idx = x.reshape(_N).astype(jnp.int32)

    gather = pl.kernel(
        _gather_body,
        out_type=jax.ShapeDtypeStruct((_NT, _B), jnp.float32),
        mesh=plsc.VectorSubcoreMesh(
            core_axis_name="core", subcore_axis_name="subcore"
        ),
        scratch_types=[
            pltpu.VMEM((_IPS,), jnp.int32),
            pltpu.VMEM((2, _BPS), jnp.int32),
            pltpu.VMEM((2, _BPS, _DP), jnp.float32),
            pltpu.VMEM((2, 96, _BPS), jnp.float32),
            pltpu.SemaphoreType.DMA,
            pltpu.SemaphoreType.DMA((2,)),
            pltpu.SemaphoreType.DMA((2,)),
        ],
        compiler_params=pltpu.CompilerParams(needs_layout_passes=False),
    )
    out_t = gather(table, idx)
    return jnp.transpose(out_t.reshape(1, _L, _D, _B), (3, 0, 1, 2))


# speedup vs baseline: 2.8550x; 2.8550x over previous
import jax
import jax.numpy as jnp
from jax.experimental import pallas as pl
from jax.experimental.pallas import tpu as pltpu
from jax.experimental.pallas import tpu_sc as plsc

_B = 4096
_L = 200
_D = 192
_DP = 256
_N = _B * _L
_W = 128
_NBUF = 2
_NSUB = 32
_WPS = _N // (_W * _NSUB)
_IPS = _N // _NSUB
_GROUPS = _WPS // _NBUF


def _gather_body(tbl_hbm, idx_hbm, out_hbm, idx_ref, s0, s1, h0, h1, si, gs, ws):
    ss = [s0, s1]
    h64s = [h0, h1]
    core = jax.lax.axis_index("core")
    sub = jax.lax.axis_index("subcore")
    sid = core * 16 + sub
    wbase = sid * _WPS

    pltpu.async_copy(idx_hbm.at[pl.ds(sid * _IPS, _IPS)], idx_ref, si).wait()

    def start_gather(w, b):
        iv = idx_ref.at[pl.ds(w * _W, _W)]
        pltpu.async_copy(tbl_hbm.at[iv], ss[b], gs.at[b])

    def wait_gather(b):
        iv = idx_ref.at[pl.ds(0, _W)]
        pltpu.make_async_copy(tbl_hbm.at[iv], ss[b], gs.at[b]).wait()

    def repack(b):
        @pl.loop(0, _W)
        def _(r):
            for j in range(4):
                src = (pl.ds(r, 1), pl.ds(128 + j * 16, 16))
                dst = (pl.ds(r, 1), pl.ds(j * 16, 16))
                h64s[b].at[dst][...] = ss[b].at[src][...]

    def start_writes(w, b):
        rows = pl.ds((wbase + w) * _W, _W)
        pltpu.async_copy(
            ss[b].at[:, pl.ds(0, 128)], out_hbm.at[rows, pl.ds(0, 128)], ws.at[b]
        )
        pltpu.async_copy(h64s[b], out_hbm.at[rows, pl.ds(128, 64)], ws.at[b])

    def wait_writes(b):
        rows = pl.ds(wbase * _W, _W)
        pltpu.make_async_copy(
            ss[b].at[:, pl.ds(0, 128)], out_hbm.at[rows, pl.ds(0, 128)], ws.at[b]
        ).wait()
        pltpu.make_async_copy(
            h64s[b], out_hbm.at[rows, pl.ds(128, 64)], ws.at[b]
        ).wait()

    for b in range(_NBUF):
        start_gather(b, b)

    @pl.loop(1, _GROUPS)
    def _(g):
        for b in range(_NBUF):
            wait_gather(b)
            repack(b)
            start_writes((g - 1) * _NBUF + b, b)
        for b in range(_NBUF):
            wait_writes(b)
            start_gather(g * _NBUF + b, b)

    for b in range(_NBUF):
        wait_gather(b)
        repack(b)
        start_writes((_GROUPS - 1) * _NBUF + b, b)
    for b in range(_NBUF):
        wait_writes(b)


def kernel(x, emb1_weight, emb2_weight):
    table = jnp.concatenate(
        (
            emb1_weight,
            emb2_weight,
            jnp.zeros((emb1_weight.shape[0], _DP - _D), emb1_weight.dtype),
        ),
        axis=1,
    )
    idx = x.reshape(_N).astype(jnp.int32)

    gather = pl.kernel(
        _gather_body,
        out_type=jax.ShapeDtypeStruct((_N, _D), jnp.float32),
        mesh=plsc.VectorSubcoreMesh(
            core_axis_name="core", subcore_axis_name="subcore"
        ),
        scratch_types=[
            pltpu.VMEM((_IPS,), jnp.int32),
            pltpu.VMEM((_W, _DP), jnp.float32),
            pltpu.VMEM((_W, _DP), jnp.float32),
            pltpu.VMEM((_W, 64), jnp.float32),
            pltpu.VMEM((_W, 64), jnp.float32),
            pltpu.SemaphoreType.DMA,
            pltpu.SemaphoreType.DMA((_NBUF,)),
            pltpu.SemaphoreType.DMA((_NBUF,)),
        ],
    )
    out = gather(table, idx)
    return out.reshape(_B, 1, _L, _D)
